# Initial kernel scaffold; baseline (speedup 1.0000x reference)
import jax
import jax.numpy as jnp
from jax.experimental import pallas as pl


def kernel(user_indices, item_indices, user_table, item_table, W1, b1, W2, b2, W3, b3, Wo, bo):
    raise NotImplementedError("write your pallas kernel here")


# speedup vs baseline: 2.5714x; 2.5714x over previous
import functools

import jax
import jax.numpy as jnp
from jax import lax
from jax.experimental import pallas as pl
from jax.experimental.pallas import tpu as pltpu
from jax.experimental.pallas import tpu_sc as plsc

_NC = 2
_NS = 16
_NW = _NC * _NS

_B = 16384
_D = 128
_CHUNK = 128
_ROWS_PER_W = _B // _NW
_CPW = _ROWS_PER_W // _CHUNK


def _gather_body(uidx_hbm, iidx_hbm, utab_hbm, itab_hbm, ue_out, ie_out,
                 idx_u, idx_i, rows, sem):
    wid = lax.axis_index("s") * _NC + lax.axis_index("c")
    base = wid * _CPW
    pltpu.sync_copy(uidx_hbm.at[pl.ds(base, _CPW)], idx_u)
    pltpu.sync_copy(iidx_hbm.at[pl.ds(base, _CPW)], idx_i)
    cps = [pltpu.async_copy(utab_hbm.at[idx_u.at[j]], rows.at[j], sem)
           for j in range(_CPW)]
    for c in cps:
        c.wait()
    pltpu.sync_copy(rows, ue_out.at[pl.ds(base, _CPW)])
    cps = [pltpu.async_copy(itab_hbm.at[idx_i.at[j]], rows.at[j], sem)
           for j in range(_CPW)]
    for c in cps:
        c.wait()
    pltpu.sync_copy(rows, ie_out.at[pl.ds(base, _CPW)])


@jax.jit
def _sc_gather(uidx2d, iidx2d, user_table, item_table):
    mesh = plsc.VectorSubcoreMesh(core_axis_name="c", subcore_axis_name="s",
                                  num_cores=_NC, num_subcores=_NS)
    grab = pl.kernel(
        _gather_body,
        out_type=[
            jax.ShapeDtypeStruct((_B // _CHUNK, _CHUNK, _D), jnp.float32),
            jax.ShapeDtypeStruct((_B // _CHUNK, _CHUNK, _D), jnp.float32),
        ],
        mesh=mesh,
        scratch_types=[
            pltpu.VMEM((_CPW, _CHUNK), jnp.int32),
            pltpu.VMEM((_CPW, _CHUNK), jnp.int32),
            pltpu.VMEM((_CPW, _CHUNK, _D), jnp.float32),
            pltpu.SemaphoreType.DMA,
        ],
        name="ncf_sc_gather",
    )
    return grab(uidx2d, iidx2d, user_table, item_table)


def _mlp_body(ue_ref, ie_ref, w1a, w1b, b1, w2, b2, w3, b3, wo, bo, out_ref):
    h = jnp.maximum(
        ue_ref[...] @ w1a[...] + ie_ref[...] @ w1b[...] + b1[...], 0.0)
    h = jnp.maximum(h @ w2[...] + b2[...], 0.0)
    h = jnp.maximum(h @ w3[...] + b3[...], 0.0)
    out_ref[...] = h @ wo[...] + bo[...]


_BM = 1024


@jax.jit
def _tc_mlp(ue, ie, w1a, w1b, b1, w2, b2, w3, b3, wo, bo):
    full = lambda shape: pl.BlockSpec(shape, lambda i: (0, 0))
    return pl.pallas_call(
        _mlp_body,
        grid=(_B // _BM,),
        in_specs=[
            pl.BlockSpec((_BM, _D), lambda i: (i, 0)),
            pl.BlockSpec((_BM, _D), lambda i: (i, 0)),
            full((128, 128)), full((128, 128)), full((1, 128)),
            full((128, 64)), full((1, 64)),
            full((64, 32)), full((1, 32)),
            full((32, 1)), full((1, 1)),
        ],
        out_specs=pl.BlockSpec((_BM, 1), lambda i: (i, 0)),
        out_shape=jax.ShapeDtypeStruct((_B, 1), jnp.float32),
        name="ncf_tc_mlp",
    )(ue, ie, w1a, w1b, b1, w2, b2, w3, b3, wo, bo)


def kernel(user_indices, item_indices, user_table, item_table,
           W1, b1, W2, b2, W3, b3, Wo, bo):
    uidx2d = user_indices.reshape(_B // _CHUNK, _CHUNK)
    iidx2d = item_indices.reshape(_B // _CHUNK, _CHUNK)
    ue3d, ie3d = _sc_gather(uidx2d, iidx2d, user_table, item_table)
    ue = ue3d.reshape(_B, _D)
    ie = ie3d.reshape(_B, _D)
    return _tc_mlp(ue, ie,
                   W1[:_D], W1[_D:], b1.reshape(1, -1),
                   W2, b2.reshape(1, -1),
                   W3, b3.reshape(1, -1),
                   Wo, bo.reshape(1, -1))
